# baseline (speedup 1.0000x reference)
import functools

import jax
import jax.numpy as jnp
from jax import lax
from jax.experimental import pallas as pl
from jax.experimental.pallas import tpu as pltpu
from jax.experimental.pallas import tpu_sc as plsc

EMBED = 64
_PAIRW = 128
_BLK = 128
_NSLOT = 3
_NOBUF = 2
_LANES = 16


@functools.lru_cache(maxsize=None)
def _make_gather(vocab: int, n_idx: int, n_i: int):
    info = plsc.get_sparse_core_info()
    nc, ns = info.num_cores, info.num_subcores
    nw = nc * ns
    assert n_idx % (nw * _BLK) == 0 and n_i % _BLK == 0
    per_w = n_idx // nw
    n_blocks = per_w // _BLK
    blocks_per_i = n_i // _BLK

    mesh = plsc.VectorSubcoreMesh(core_axis_name="c", subcore_axis_name="s")

    @functools.partial(
        pl.kernel,
        mesh=mesh,
        out_type=jax.ShapeDtypeStruct((n_idx // n_i * EMBED, n_i),
                                      jnp.float32),
        scratch_types=[
            pltpu.VMEM((per_w,), jnp.int32),
            pltpu.VMEM((per_w,), jnp.int32),
            pltpu.VMEM((_NSLOT * _BLK, _PAIRW), jnp.float32),
            pltpu.VMEM((_NOBUF * EMBED, _BLK), jnp.float32),
            pltpu.SemaphoreType.DMA,
            pltpu.SemaphoreType.DMA,
        ],
        compiler_params=pltpu.CompilerParams(needs_layout_passes=False),
    )
    def gather(table_hbm, idx_hbm, out_hbm, p_v, h_v, rows_v, obuf, g_sem,
               w_sem):
        wid = lax.axis_index("s") * nc + lax.axis_index("c")
        base = wid * per_w

        pltpu.sync_copy(idx_hbm.at[pl.ds(base, per_w)], p_v)

        def prep(k, carry):
            v = p_v[pl.ds(k * _LANES, _LANES)]
            p_v[pl.ds(k * _LANES, _LANES)] = lax.shift_right_logical(v, 1)
            h_v[pl.ds(k * _LANES, _LANES)] = lax.shift_left(
                lax.bitwise_and(v, 1), 6)
            return carry

        lax.fori_loop(0, per_w // _LANES, prep, 0)

        def fire_gather(b, slot):
            pltpu.async_copy(
                table_hbm.at[p_v.at[pl.ds(b * _BLK, _BLK)]],
                rows_v.at[pl.ds(slot * _BLK, _BLK)],
                g_sem,
            )

        def drain_gather():
            pltpu.make_async_copy(
                table_hbm.at[pl.ds(0, _BLK)],
                rows_v.at[pl.ds(0, _BLK)],
                g_sem,
            ).wait()

        def drain_write():
            pltpu.make_async_copy(
                obuf.at[pl.ds(0, EMBED)],
                out_hbm.at[pl.ds(0, EMBED), pl.ds(0, _BLK)],
                w_sem,
            ).wait()

        groups = _BLK // _LANES

        def shuffle(b, slot, os):
            def dstep(d, carry):
                iota = lax.iota(jnp.int32, _LANES)
                srow = jnp.full((_LANES,), os * EMBED + d, jnp.int32)
                for g2 in range(groups):
                    rowv = iota + (slot * _BLK + g2 * _LANES)
                    colv = h_v[pl.ds(b * _BLK + g2 * _LANES, _LANES)] + d
                    r = plsc.load_gather(rows_v, [rowv, colv])
                    plsc.store_scatter(obuf, [srow, iota + g2 * _LANES], r)
                return carry

            lax.fori_loop(0, EMBED, dstep, 0)

        fire_gather(0, 0)

        def step(b, carry):
            slot = lax.rem(b, _NSLOT)
            os = lax.rem(b, _NOBUF)

            @pl.when(b >= _NOBUF)
            def _():
                drain_write()

            @pl.when(b + 1 < n_blocks)
            def _():
                fire_gather(b + 1, lax.rem(b + 1, _NSLOT))

            drain_gather()
            shuffle(b, slot, os)
            jb = wid * n_blocks + b
            j = lax.div(jb, blocks_per_i)
            i0 = lax.rem(jb, blocks_per_i) * _BLK
            pltpu.async_copy(
                obuf.at[pl.ds(os * EMBED, EMBED)],
                out_hbm.at[pl.ds(j * EMBED, EMBED), pl.ds(i0, _BLK)],
                w_sem,
            )
            return carry

        lax.fori_loop(0, n_blocks, step, 0)
        for _ in range(min(_NOBUF, n_blocks)):
            drain_write()

    return gather


def kernel(x, word_embeddings):
    n_i, n_j = x.shape
    vocab = word_embeddings.shape[0]
    tbl2 = word_embeddings.reshape(vocab // 2, _PAIRW)
    xt = x.T.reshape(-1).astype(jnp.int32)
    out = _make_gather(vocab, x.size, n_i)(tbl2, xt)
    return out.reshape(n_j, EMBED, n_i).transpose(2, 0, 1)


# speedup vs baseline: 1.5244x; 1.5244x over previous
import functools

import jax
import jax.numpy as jnp
from jax import lax
from jax.experimental import pallas as pl
from jax.experimental.pallas import tpu as pltpu
from jax.experimental.pallas import tpu_sc as plsc

EMBED = 64
_PAIRW = 128
_BLK = 128
_NSLOT = 3
_NOBUF = 2
_LANES = 16


@functools.lru_cache(maxsize=None)
def _make_gather(vocab: int, n_idx: int, n_i: int):
    info = plsc.get_sparse_core_info()
    nc, ns = info.num_cores, info.num_subcores
    nw = nc * ns
    assert n_idx % (nw * _BLK) == 0 and n_i % _BLK == 0
    per_w = n_idx // nw
    n_blocks = per_w // _BLK
    blocks_per_i = n_i // _BLK

    mesh = plsc.VectorSubcoreMesh(core_axis_name="c", subcore_axis_name="s")

    @functools.partial(
        pl.kernel,
        mesh=mesh,
        out_type=jax.ShapeDtypeStruct((n_idx // n_i * EMBED, n_i),
                                      jnp.float32),
        scratch_types=[
            pltpu.VMEM((per_w,), jnp.int32),
            pltpu.VMEM((per_w,), jnp.int32),
            pltpu.VMEM((_NSLOT * _BLK, _PAIRW), jnp.float32),
            pltpu.VMEM((_NOBUF * EMBED, _BLK), jnp.float32),
            pltpu.SemaphoreType.DMA,
            pltpu.SemaphoreType.DMA,
        ],
        compiler_params=pltpu.CompilerParams(needs_layout_passes=False),
    )
    def gather(table_hbm, idx_hbm, out_hbm, p_v, h_v, rows_v, obuf, g_sem,
               w_sem):
        wid = lax.axis_index("s") * nc + lax.axis_index("c")
        base = wid * per_w

        pltpu.sync_copy(idx_hbm.at[pl.ds(base, per_w)], p_v)

        def prep(k, carry):
            v = p_v[pl.ds(k * _LANES, _LANES)]
            p_v[pl.ds(k * _LANES, _LANES)] = lax.shift_right_logical(v, 1)
            h_v[pl.ds(k * _LANES, _LANES)] = lax.shift_left(
                lax.bitwise_and(v, 1), 6)
            return carry

        lax.fori_loop(0, per_w // _LANES, prep, 0)

        def fire_gather(b, slot):
            pltpu.async_copy(
                table_hbm.at[p_v.at[pl.ds(b * _BLK, _BLK)]],
                rows_v.at[pl.ds(slot * _BLK, _BLK)],
                g_sem,
            )

        def drain_gather():
            pltpu.make_async_copy(
                table_hbm.at[pl.ds(0, _BLK)],
                rows_v.at[pl.ds(0, _BLK)],
                g_sem,
            ).wait()

        def drain_write():
            pltpu.make_async_copy(
                obuf.at[pl.ds(0, EMBED)],
                out_hbm.at[pl.ds(0, EMBED), pl.ds(0, _BLK)],
                w_sem,
            ).wait()

        groups = _BLK // _LANES
        _UNROLL = 8

        def shuffle(b, slot, os):
            iota = lax.iota(jnp.int32, _LANES)
            rowvs = [iota + (slot * _BLK + g2 * _LANES)
                     for g2 in range(groups)]
            colvs = [h_v[pl.ds(b * _BLK + g2 * _LANES, _LANES)]
                     for g2 in range(groups)]
            scols = [iota + g2 * _LANES for g2 in range(groups)]

            def dstep(du, carry):
                for dd in range(_UNROLL):
                    d = du * _UNROLL + dd
                    srow = jnp.full((_LANES,), os * EMBED + d, jnp.int32)
                    for g2 in range(groups):
                        r = plsc.load_gather(rows_v,
                                             [rowvs[g2], colvs[g2] + d])
                        plsc.store_scatter(obuf, [srow, scols[g2]], r)
                return carry

            lax.fori_loop(0, EMBED // _UNROLL, dstep, 0)

        fire_gather(0, 0)

        def step(b, carry):
            slot = lax.rem(b, _NSLOT)
            os = lax.rem(b, _NOBUF)

            @pl.when(b >= _NOBUF)
            def _():
                drain_write()

            @pl.when(b + 1 < n_blocks)
            def _():
                fire_gather(b + 1, lax.rem(b + 1, _NSLOT))

            drain_gather()
            shuffle(b, slot, os)
            jb = wid * n_blocks + b
            j = lax.div(jb, blocks_per_i)
            i0 = lax.rem(jb, blocks_per_i) * _BLK
            pltpu.async_copy(
                obuf.at[pl.ds(os * EMBED, EMBED)],
                out_hbm.at[pl.ds(j * EMBED, EMBED), pl.ds(i0, _BLK)],
                w_sem,
            )
            return carry

        lax.fori_loop(0, n_blocks, step, 0)
        for _ in range(min(_NOBUF, n_blocks)):
            drain_write()

    return gather


def kernel(x, word_embeddings):
    n_i, n_j = x.shape
    vocab = word_embeddings.shape[0]
    tbl2 = word_embeddings.reshape(vocab // 2, _PAIRW)
    xt = x.T.reshape(-1).astype(jnp.int32)
    out = _make_gather(vocab, x.size, n_i)(tbl2, xt)
    return out.reshape(n_j, EMBED, n_i).transpose(2, 0, 1)


# speedup vs baseline: 1.5272x; 1.0018x over previous
import functools

import jax
import jax.numpy as jnp
from jax import lax
from jax.experimental import pallas as pl
from jax.experimental.pallas import tpu as pltpu
from jax.experimental.pallas import tpu_sc as plsc

EMBED = 64
_PAIRW = 128
_BLK = 128
_NSLOT = 3
_NOBUF = 2
_LANES = 16


@functools.lru_cache(maxsize=None)
def _make_gather(vocab: int, n_idx: int, n_i: int):
    info = plsc.get_sparse_core_info()
    nc, ns = info.num_cores, info.num_subcores
    nw = nc * ns
    assert n_idx % (nw * _BLK) == 0 and n_i % _BLK == 0
    per_w = n_idx // nw
    n_blocks = per_w // _BLK
    blocks_per_i = n_i // _BLK

    mesh = plsc.VectorSubcoreMesh(core_axis_name="c", subcore_axis_name="s")

    @functools.partial(
        pl.kernel,
        mesh=mesh,
        out_type=jax.ShapeDtypeStruct((n_idx // n_i * EMBED, n_i),
                                      jnp.float32),
        scratch_types=[
            pltpu.VMEM((per_w,), jnp.int32),
            pltpu.VMEM((per_w,), jnp.int32),
            pltpu.VMEM((_NSLOT * _BLK, _PAIRW), jnp.float32),
            pltpu.VMEM((_NOBUF * EMBED, _BLK), jnp.float32),
            pltpu.SemaphoreType.DMA,
            pltpu.SemaphoreType.DMA,
        ],
        compiler_params=pltpu.CompilerParams(needs_layout_passes=False),
    )
    def gather(table_hbm, idx_hbm, out_hbm, p_v, h_v, rows_v, obuf, g_sem,
               w_sem):
        wid = lax.axis_index("s") * nc + lax.axis_index("c")
        base = wid * per_w

        pltpu.sync_copy(idx_hbm.at[pl.ds(base, per_w)], p_v)

        def prep(k, carry):
            v = p_v[pl.ds(k * _LANES, _LANES)]
            p_v[pl.ds(k * _LANES, _LANES)] = lax.shift_right_logical(v, 1)
            h_v[pl.ds(k * _LANES, _LANES)] = lax.shift_left(
                lax.bitwise_and(v, 1), 6)
            return carry

        lax.fori_loop(0, per_w // _LANES, prep, 0)

        def fire_gather(b, slot):
            pltpu.async_copy(
                table_hbm.at[p_v.at[pl.ds(b * _BLK, _BLK)]],
                rows_v.at[pl.ds(slot * _BLK, _BLK)],
                g_sem,
            )

        def drain_gather():
            pltpu.make_async_copy(
                table_hbm.at[pl.ds(0, _BLK)],
                rows_v.at[pl.ds(0, _BLK)],
                g_sem,
            ).wait()

        def drain_write():
            pltpu.make_async_copy(
                obuf.at[pl.ds(0, EMBED)],
                out_hbm.at[pl.ds(0, EMBED), pl.ds(0, _BLK)],
                w_sem,
            ).wait()

        groups = _BLK // _LANES
        _UNROLL = 8

        def shuffle(b, slot, os):
            iota = lax.iota(jnp.int32, _LANES)
            rowvs = [iota + (slot * _BLK + g2 * _LANES)
                     for g2 in range(groups)]
            colvs = [h_v[pl.ds(b * _BLK + g2 * _LANES, _LANES)]
                     for g2 in range(groups)]
            scols = [iota + g2 * _LANES for g2 in range(groups)]

            def dstep(du, carry):
                for dd in range(_UNROLL):
                    d = du * _UNROLL + dd
                    orow = os * EMBED + d
                    for g2 in range(groups):
                        r = plsc.load_gather(rows_v,
                                             [rowvs[g2], colvs[g2] + d])
                        obuf[orow, pl.ds(g2 * _LANES, _LANES)] = r
                return carry

            lax.fori_loop(0, EMBED // _UNROLL, dstep, 0)

        fire_gather(0, 0)

        def step(b, carry):
            slot = lax.rem(b, _NSLOT)
            os = lax.rem(b, _NOBUF)

            @pl.when(b >= _NOBUF)
            def _():
                drain_write()

            @pl.when(b + 1 < n_blocks)
            def _():
                fire_gather(b + 1, lax.rem(b + 1, _NSLOT))

            drain_gather()
            shuffle(b, slot, os)
            jb = wid * n_blocks + b
            j = lax.div(jb, blocks_per_i)
            i0 = lax.rem(jb, blocks_per_i) * _BLK
            pltpu.async_copy(
                obuf.at[pl.ds(os * EMBED, EMBED)],
                out_hbm.at[pl.ds(j * EMBED, EMBED), pl.ds(i0, _BLK)],
                w_sem,
            )
            return carry

        lax.fori_loop(0, n_blocks, step, 0)
        for _ in range(min(_NOBUF, n_blocks)):
            drain_write()

    return gather


def kernel(x, word_embeddings):
    n_i, n_j = x.shape
    vocab = word_embeddings.shape[0]
    tbl2 = word_embeddings.reshape(vocab // 2, _PAIRW)
    xt = x.T.reshape(-1).astype(jnp.int32)
    out = _make_gather(vocab, x.size, n_i)(tbl2, xt)
    return out.reshape(n_j, EMBED, n_i).transpose(2, 0, 1)


# speedup vs baseline: 2.0289x; 1.3285x over previous
import functools

import jax
import jax.numpy as jnp
from jax import lax
from jax.experimental import pallas as pl
from jax.experimental.pallas import tpu as pltpu
from jax.experimental.pallas import tpu_sc as plsc

EMBED = 64
_PAIRW = 128
_BLK = 128
_NSLOT = 3
_NOBUF = 2
_LANES = 16


@functools.lru_cache(maxsize=None)
def _make_gather(vocab: int, n_idx: int, n_i: int):
    info = plsc.get_sparse_core_info()
    nc, ns = info.num_cores, info.num_subcores
    nw = nc * ns
    assert n_idx % (nw * _BLK) == 0 and n_i % _BLK == 0
    per_w = n_idx // nw
    n_blocks = per_w // _BLK
    blocks_per_i = n_i // _BLK

    mesh = plsc.VectorSubcoreMesh(core_axis_name="c", subcore_axis_name="s")

    @functools.partial(
        pl.kernel,
        mesh=mesh,
        out_type=jax.ShapeDtypeStruct((n_idx // n_i * EMBED, n_i),
                                      jnp.float32),
        scratch_types=[
            pltpu.VMEM((per_w,), jnp.int32),
            pltpu.VMEM((per_w,), jnp.int32),
            pltpu.VMEM((_NSLOT * _BLK, _PAIRW), jnp.float32),
            pltpu.VMEM((_NOBUF * EMBED, _BLK), jnp.float32),
            pltpu.SemaphoreType.DMA,
            pltpu.SemaphoreType.DMA,
        ],
        compiler_params=pltpu.CompilerParams(needs_layout_passes=False),
    )
    def gather(table_hbm, idx_hbm, out_hbm, p_v, h_v, rows_v, obuf, g_sem,
               w_sem):
        wid = lax.axis_index("s") * nc + lax.axis_index("c")
        base = wid * per_w

        pltpu.sync_copy(idx_hbm.at[pl.ds(base, per_w)], p_v)

        @plsc.parallel_loop(0, per_w // _LANES, step=1, unroll=4)
        def _prep(k):
            v = p_v[pl.ds(k * _LANES, _LANES)]
            p_v[pl.ds(k * _LANES, _LANES)] = lax.shift_right_logical(v, 1)
            h_v[pl.ds(k * _LANES, _LANES)] = lax.shift_left(
                lax.bitwise_and(v, 1), 6)

        def fire_gather(b, slot):
            pltpu.async_copy(
                table_hbm.at[p_v.at[pl.ds(b * _BLK, _BLK)]],
                rows_v.at[pl.ds(slot * _BLK, _BLK)],
                g_sem,
            )

        def drain_gather():
            pltpu.make_async_copy(
                table_hbm.at[pl.ds(0, _BLK)],
                rows_v.at[pl.ds(0, _BLK)],
                g_sem,
            ).wait()

        def drain_write():
            pltpu.make_async_copy(
                obuf.at[pl.ds(0, EMBED)],
                out_hbm.at[pl.ds(0, EMBED), pl.ds(0, _BLK)],
                w_sem,
            ).wait()

        groups = _BLK // _LANES
        _UNROLL = 8

        def shuffle(b, slot, os):
            iota = lax.iota(jnp.int32, _LANES)
            rowvs = [iota + (slot * _BLK + g2 * _LANES)
                     for g2 in range(groups)]
            colvs = [h_v[pl.ds(b * _BLK + g2 * _LANES, _LANES)]
                     for g2 in range(groups)]

            @plsc.parallel_loop(0, EMBED, step=1, unroll=_UNROLL)
            def _dstep(d):
                orow = os * EMBED + d
                for g2 in range(groups):
                    r = plsc.load_gather(rows_v,
                                         [rowvs[g2], colvs[g2] + d])
                    obuf[orow, pl.ds(g2 * _LANES, _LANES)] = r

        fire_gather(0, 0)

        def step(b, carry):
            slot = lax.rem(b, _NSLOT)
            os = lax.rem(b, _NOBUF)

            @pl.when(b >= _NOBUF)
            def _():
                drain_write()

            @pl.when(b + 1 < n_blocks)
            def _():
                fire_gather(b + 1, lax.rem(b + 1, _NSLOT))

            drain_gather()
            shuffle(b, slot, os)
            jb = wid * n_blocks + b
            j = lax.div(jb, blocks_per_i)
            i0 = lax.rem(jb, blocks_per_i) * _BLK
            pltpu.async_copy(
                obuf.at[pl.ds(os * EMBED, EMBED)],
                out_hbm.at[pl.ds(j * EMBED, EMBED), pl.ds(i0, _BLK)],
                w_sem,
            )
            return carry

        lax.fori_loop(0, n_blocks, step, 0)
        for _ in range(min(_NOBUF, n_blocks)):
            drain_write()

    return gather


def kernel(x, word_embeddings):
    n_i, n_j = x.shape
    vocab = word_embeddings.shape[0]
    tbl2 = word_embeddings.reshape(vocab // 2, _PAIRW)
    xt = x.T.reshape(-1).astype(jnp.int32)
    out = _make_gather(vocab, x.size, n_i)(tbl2, xt)
    return out.reshape(n_j, EMBED, n_i).transpose(2, 0, 1)


# speedup vs baseline: 3.2163x; 1.5852x over previous
import functools

import jax
import jax.numpy as jnp
from jax import lax
from jax.experimental import pallas as pl
from jax.experimental.pallas import tpu as pltpu
from jax.experimental.pallas import tpu_sc as plsc

EMBED = 64
_PAIRW = 128
_BLK = 128
_NSLOT = 3
_NOBUF = 2
_LANES = 16


@functools.lru_cache(maxsize=None)
def _make_gather(vocab: int, n_idx: int, n_i: int):
    info = plsc.get_sparse_core_info()
    nc, ns = info.num_cores, info.num_subcores
    nw = nc * ns
    assert n_idx % (nw * _BLK) == 0 and n_i % _BLK == 0
    per_w = n_idx // nw
    n_blocks = per_w // _BLK
    blocks_per_i = n_i // _BLK

    mesh = plsc.VectorSubcoreMesh(core_axis_name="c", subcore_axis_name="s")

    @functools.partial(
        pl.kernel,
        mesh=mesh,
        out_type=jax.ShapeDtypeStruct((n_idx // n_i * EMBED, n_i),
                                      jnp.float32),
        scratch_types=[
            pltpu.VMEM((per_w,), jnp.int32),
            pltpu.VMEM((per_w,), jnp.int32),
            pltpu.VMEM((_NSLOT * _BLK, _PAIRW), jnp.float32),
            pltpu.VMEM((_NOBUF * EMBED, _BLK), jnp.float32),
            pltpu.SemaphoreType.DMA,
            pltpu.SemaphoreType.DMA,
        ],
        compiler_params=pltpu.CompilerParams(needs_layout_passes=False),
    )
    def gather(table_hbm, idx_hbm, out_hbm, p_v, h_v, rows_v, obuf, g_sem,
               w_sem):
        wid = lax.axis_index("s") * nc + lax.axis_index("c")
        base = wid * per_w

        pltpu.sync_copy(idx_hbm.at[pl.ds(base, per_w)], p_v)

        @plsc.parallel_loop(0, per_w // _LANES, step=1, unroll=4)
        def _prep(k):
            v = p_v[pl.ds(k * _LANES, _LANES)]
            p_v[pl.ds(k * _LANES, _LANES)] = lax.shift_right_logical(v, 1)
            h_v[pl.ds(k * _LANES, _LANES)] = lax.shift_left(
                lax.bitwise_and(v, 1), 6)

        def fire_gather(b, slot):
            pltpu.async_copy(
                table_hbm.at[p_v.at[pl.ds(b * _BLK, _BLK)]],
                rows_v.at[pl.ds(slot * _BLK, _BLK)],
                g_sem,
            )

        def drain_gather():
            pltpu.make_async_copy(
                table_hbm.at[pl.ds(0, _BLK)],
                rows_v.at[pl.ds(0, _BLK)],
                g_sem,
            ).wait()

        def drain_write():
            pltpu.make_async_copy(
                obuf.at[pl.ds(0, EMBED)],
                out_hbm.at[pl.ds(0, EMBED), pl.ds(0, _BLK)],
                w_sem,
            ).wait()

        groups = _BLK // _LANES
        _UNROLL = 8

        def shuffle(b, slot, os):
            iota = lax.iota(jnp.int32, _LANES)
            rowvs = [iota + (slot * _BLK + g2 * _LANES)
                     for g2 in range(groups)]
            colvs = [h_v[pl.ds(b * _BLK + g2 * _LANES, _LANES)]
                     for g2 in range(groups)]
            scols = [iota + g2 * _LANES for g2 in range(groups)]

            @plsc.parallel_loop(0, EMBED, step=1, unroll=_UNROLL)
            def _dstep(d):
                t = lax.bitwise_and(iota + d, EMBED - 1)
                orows = t + os * EMBED
                for g2 in range(groups):
                    r = plsc.load_gather(rows_v, [rowvs[g2], colvs[g2] + t])
                    plsc.store_scatter(obuf, [orows, scols[g2]], r)

        fire_gather(0, 0)

        def step(b, carry):
            slot = lax.rem(b, _NSLOT)
            os = lax.rem(b, _NOBUF)

            @pl.when(b >= _NOBUF)
            def _():
                drain_write()

            @pl.when(b + 1 < n_blocks)
            def _():
                fire_gather(b + 1, lax.rem(b + 1, _NSLOT))

            drain_gather()
            shuffle(b, slot, os)
            jb = wid * n_blocks + b
            j = lax.div(jb, blocks_per_i)
            i0 = lax.rem(jb, blocks_per_i) * _BLK
            pltpu.async_copy(
                obuf.at[pl.ds(os * EMBED, EMBED)],
                out_hbm.at[pl.ds(j * EMBED, EMBED), pl.ds(i0, _BLK)],
                w_sem,
            )
            return carry

        lax.fori_loop(0, n_blocks, step, 0)
        for _ in range(min(_NOBUF, n_blocks)):
            drain_write()

    return gather


def kernel(x, word_embeddings):
    n_i, n_j = x.shape
    vocab = word_embeddings.shape[0]
    tbl2 = word_embeddings.reshape(vocab // 2, _PAIRW)
    xt = x.T.reshape(-1).astype(jnp.int32)
    out = _make_gather(vocab, x.size, n_i)(tbl2, xt)
    return out.reshape(n_j, EMBED, n_i).transpose(2, 0, 1)


# speedup vs baseline: 3.2225x; 1.0019x over previous
import functools

import jax
import jax.numpy as jnp
from jax import lax
from jax.experimental import pallas as pl
from jax.experimental.pallas import tpu as pltpu
from jax.experimental.pallas import tpu_sc as plsc

EMBED = 64
_PAIRW = 128
_BLK = 128
_NSLOT = 3
_NOBUF = 2
_LANES = 16


@functools.lru_cache(maxsize=None)
def _make_gather(vocab: int, n_idx: int, n_i: int):
    info = plsc.get_sparse_core_info()
    nc, ns = info.num_cores, info.num_subcores
    nw = nc * ns
    assert n_idx % (nw * _BLK) == 0 and n_i % _BLK == 0
    per_w = n_idx // nw
    n_blocks = per_w // _BLK
    blocks_per_i = n_i // _BLK

    mesh = plsc.VectorSubcoreMesh(core_axis_name="c", subcore_axis_name="s")

    @functools.partial(
        pl.kernel,
        mesh=mesh,
        out_type=jax.ShapeDtypeStruct((n_idx // n_i * EMBED, n_i),
                                      jnp.float32),
        scratch_types=[
            pltpu.VMEM((per_w,), jnp.int32),
            pltpu.VMEM((per_w,), jnp.int32),
            pltpu.VMEM((_NSLOT * _BLK, _PAIRW), jnp.float32),
            pltpu.VMEM((_NOBUF * EMBED, _BLK), jnp.float32),
            pltpu.SemaphoreType.DMA,
            pltpu.SemaphoreType.DMA,
        ],
        compiler_params=pltpu.CompilerParams(needs_layout_passes=False),
    )
    def gather(table_hbm, idx_hbm, out_hbm, p_v, h_v, rows_v, obuf, g_sem,
               w_sem):
        wid = lax.axis_index("s") * nc + lax.axis_index("c")
        base = wid * per_w

        pltpu.sync_copy(idx_hbm.at[pl.ds(base, per_w)], p_v)

        @plsc.parallel_loop(0, per_w // _LANES, step=1, unroll=4)
        def _prep(k):
            v = p_v[pl.ds(k * _LANES, _LANES)]
            p_v[pl.ds(k * _LANES, _LANES)] = lax.shift_right_logical(v, 1)
            h_v[pl.ds(k * _LANES, _LANES)] = lax.shift_left(
                lax.bitwise_and(v, 1), 6)

        def fire_gather(b, slot):
            pltpu.async_copy(
                table_hbm.at[p_v.at[pl.ds(b * _BLK, _BLK)]],
                rows_v.at[pl.ds(slot * _BLK, _BLK)],
                g_sem,
            )

        def drain_gather():
            pltpu.make_async_copy(
                table_hbm.at[pl.ds(0, _BLK)],
                rows_v.at[pl.ds(0, _BLK)],
                g_sem,
            ).wait()

        def drain_write():
            pltpu.make_async_copy(
                obuf.at[pl.ds(0, EMBED)],
                out_hbm.at[pl.ds(0, EMBED), pl.ds(0, _BLK)],
                w_sem,
            ).wait()

        groups = _BLK // _LANES
        _UNROLL = 16

        def shuffle(b, slot, os):
            iota = lax.iota(jnp.int32, _LANES)
            rowvs = [iota + (slot * _BLK + g2 * _LANES)
                     for g2 in range(groups)]
            colvs = [h_v[pl.ds(b * _BLK + g2 * _LANES, _LANES)]
                     for g2 in range(groups)]
            scols = [iota + g2 * _LANES for g2 in range(groups)]

            @plsc.parallel_loop(0, EMBED, step=1, unroll=_UNROLL)
            def _dstep(d):
                t = lax.bitwise_and(iota + d, EMBED - 1)
                orows = t + os * EMBED
                for g2 in range(groups):
                    r = plsc.load_gather(rows_v, [rowvs[g2], colvs[g2] + t])
                    plsc.store_scatter(obuf, [orows, scols[g2]], r)

        fire_gather(0, 0)

        def step(b, carry):
            slot = lax.rem(b, _NSLOT)
            os = lax.rem(b, _NOBUF)

            @pl.when(b >= _NOBUF)
            def _():
                drain_write()

            @pl.when(b + 1 < n_blocks)
            def _():
                fire_gather(b + 1, lax.rem(b + 1, _NSLOT))

            drain_gather()
            shuffle(b, slot, os)
            jb = wid * n_blocks + b
            j = lax.div(jb, blocks_per_i)
            i0 = lax.rem(jb, blocks_per_i) * _BLK
            pltpu.async_copy(
                obuf.at[pl.ds(os * EMBED, EMBED)],
                out_hbm.at[pl.ds(j * EMBED, EMBED), pl.ds(i0, _BLK)],
                w_sem,
            )
            return carry

        lax.fori_loop(0, n_blocks, step, 0)
        for _ in range(min(_NOBUF, n_blocks)):
            drain_write()

    return gather


def kernel(x, word_embeddings):
    n_i, n_j = x.shape
    vocab = word_embeddings.shape[0]
    tbl2 = word_embeddings.reshape(vocab // 2, _PAIRW)
    xt = x.T.reshape(-1).astype(jnp.int32)
    out = _make_gather(vocab, x.size, n_i)(tbl2, xt)
    return out.reshape(n_j, EMBED, n_i).transpose(2, 0, 1)


# speedup vs baseline: 3.3760x; 1.0476x over previous
import functools

import jax
import jax.numpy as jnp
from jax import lax
from jax.experimental import pallas as pl
from jax.experimental.pallas import tpu as pltpu
from jax.experimental.pallas import tpu_sc as plsc

EMBED = 64
_PAIRW = 128
_BLK = 128
_NSLOT = 3
_NOBUF = 2
_LANES = 16


@functools.lru_cache(maxsize=None)
def _make_gather(vocab: int, n_idx: int, n_i: int):
    info = plsc.get_sparse_core_info()
    nc, ns = info.num_cores, info.num_subcores
    nw = nc * ns
    assert n_idx % (nw * _BLK) == 0 and n_i % _BLK == 0
    per_w = n_idx // nw
    n_blocks = per_w // _BLK
    blocks_per_i = n_i // _BLK

    mesh = plsc.VectorSubcoreMesh(core_axis_name="c", subcore_axis_name="s")

    @functools.partial(
        pl.kernel,
        mesh=mesh,
        out_type=jax.ShapeDtypeStruct((n_idx // n_i * EMBED, n_i),
                                      jnp.float32),
        scratch_types=[
            pltpu.VMEM((per_w,), jnp.int32),
            pltpu.VMEM((per_w,), jnp.int32),
            pltpu.VMEM((_NSLOT * _BLK, _PAIRW), jnp.float32),
            pltpu.VMEM((_NOBUF * EMBED, _BLK), jnp.float32),
            pltpu.SemaphoreType.DMA,
            pltpu.SemaphoreType.DMA,
        ],
        compiler_params=pltpu.CompilerParams(needs_layout_passes=False),
    )
    def gather(table_hbm, idx_hbm, out_hbm, p_v, h_v, rows_v, obuf, g_sem,
               w_sem):
        wid = lax.axis_index("s") * nc + lax.axis_index("c")
        base = wid * per_w

        pltpu.sync_copy(idx_hbm.at[pl.ds(base, per_w)], p_v)

        @plsc.parallel_loop(0, per_w // _LANES, step=1, unroll=4)
        def _prep(k):
            v = p_v[pl.ds(k * _LANES, _LANES)]
            p_v[pl.ds(k * _LANES, _LANES)] = lax.shift_right_logical(v, 1)
            h_v[pl.ds(k * _LANES, _LANES)] = lax.shift_left(
                lax.bitwise_and(v, 1), 6)

        def fire_gather(b, slot):
            pltpu.async_copy(
                table_hbm.at[p_v.at[pl.ds(b * _BLK, _BLK)]],
                rows_v.at[pl.ds(slot * _BLK, _BLK)],
                g_sem,
            )

        def drain_gather():
            pltpu.make_async_copy(
                table_hbm.at[pl.ds(0, _BLK)],
                rows_v.at[pl.ds(0, _BLK)],
                g_sem,
            ).wait()

        def drain_write():
            pltpu.make_async_copy(
                obuf.at[pl.ds(0, EMBED)],
                out_hbm.at[pl.ds(0, EMBED), pl.ds(0, _BLK)],
                w_sem,
            ).wait()

        groups = _BLK // _LANES
        _UNROLL = 16

        def shuffle(b, slot, os):
            iota = lax.iota(jnp.int32, _LANES)
            rowvs = [iota + (slot * _BLK + g2 * _LANES)
                     for g2 in range(groups)]
            colvs = [h_v[pl.ds(b * _BLK + g2 * _LANES, _LANES)]
                     for g2 in range(groups)]
            scols = [iota + g2 * _LANES for g2 in range(groups)]

            @plsc.parallel_loop(0, EMBED, step=1, unroll=_UNROLL)
            def _dstep(d):
                t = lax.bitwise_and(iota + d, EMBED - 1)
                orows = t + os * EMBED
                for g2 in range(groups):
                    r = plsc.load_gather(rows_v, [rowvs[g2], colvs[g2] + t])
                    plsc.store_scatter(obuf, [orows, scols[g2]], r)

        fire_gather(0, 0)
        fire_gather(1, 1)

        def step(b, carry):
            slot = lax.rem(b, _NSLOT)
            os = lax.rem(b, _NOBUF)

            @pl.when(b >= _NOBUF)
            def _():
                drain_write()

            @pl.when(b + 2 < n_blocks)
            def _():
                fire_gather(b + 2, lax.rem(b + 2, _NSLOT))

            drain_gather()
            shuffle(b, slot, os)
            jb = wid * n_blocks + b
            j = lax.div(jb, blocks_per_i)
            i0 = lax.rem(jb, blocks_per_i) * _BLK
            pltpu.async_copy(
                obuf.at[pl.ds(os * EMBED, EMBED)],
                out_hbm.at[pl.ds(j * EMBED, EMBED), pl.ds(i0, _BLK)],
                w_sem,
            )
            return carry

        lax.fori_loop(0, n_blocks, step, 0)
        for _ in range(min(_NOBUF, n_blocks)):
            drain_write()

    return gather


def kernel(x, word_embeddings):
    n_i, n_j = x.shape
    vocab = word_embeddings.shape[0]
    tbl2 = word_embeddings.reshape(vocab // 2, _PAIRW)
    xt = x.T.reshape(-1).astype(jnp.int32)
    out = _make_gather(vocab, x.size, n_i)(tbl2, xt)
    return out.reshape(n_j, EMBED, n_i).transpose(2, 0, 1)
